# baseline (speedup 1.0000x reference)
import functools

import jax
import jax.numpy as jnp
from jax import lax
from jax.experimental import pallas as pl
from jax.experimental.pallas import tpu as pltpu
from jax.experimental.pallas import tpu_sc as plsc

_NC = 2
_NS = 16
_NW = _NC * _NS
_LANES = 128
_GROUPS = 4
_CHUNK = _GROUPS * _LANES
_NBUF = 2


@functools.cache
def _build(n_rows, vocab, dim):
    assert n_rows % (_NW * _CHUNK) == 0
    rows_per_worker = n_rows // _NW
    chunks_per_worker = rows_per_worker // _CHUNK
    groups_per_worker = rows_per_worker // _LANES
    assert chunks_per_worker % _NBUF == 0

    mesh = plsc.VectorSubcoreMesh(core_axis_name="c", subcore_axis_name="s")

    @functools.partial(
        pl.kernel,
        out_type=jax.ShapeDtypeStruct((n_rows, dim), jnp.float32),
        mesh=mesh,
        compiler_params=pltpu.CompilerParams(use_tc_tiling_on_sc=False),
        scratch_types=[
            pltpu.VMEM((_NBUF, _GROUPS, _LANES), jnp.int32),
            pltpu.VMEM((_NBUF, _CHUNK, dim), jnp.float32),
            pltpu.SemaphoreType.DMA,
            pltpu.SemaphoreType.DMA,
            pltpu.SemaphoreType.DMA,
        ],
    )
    def gather_kernel(idx_hbm, table_hbm, out_hbm, idx_v, rows_v,
                      gsem, osem0, osem1):
        c = lax.axis_index("c")
        s = lax.axis_index("s")
        wid = s * _NC + c
        grp0 = wid * groups_per_worker
        row0 = wid * rows_per_worker
        osems = (osem0, osem1)

        def do_chunk(g, b):
            gbase = grp0 + g * _GROUPS
            rbase = row0 + g * _CHUNK
            pltpu.sync_copy(idx_hbm.at[pl.ds(gbase, _GROUPS)], idx_v.at[b])

            @pl.when(g >= _NBUF)
            def _():
                pltpu.make_async_copy(
                    rows_v.at[b], out_hbm.at[pl.ds(rbase, _CHUNK)], osems[b]
                ).wait()

            handles = [
                pltpu.async_copy(
                    table_hbm.at[idx_v.at[b, j]],
                    rows_v.at[b, pl.ds(j * _LANES, _LANES)],
                    gsem,
                )
                for j in range(_GROUPS)
            ]
            for h in handles:
                h.wait()

            pltpu.async_copy(
                rows_v.at[b], out_hbm.at[pl.ds(rbase, _CHUNK)], osems[b]
            )

        def loop_body(t, carry):
            for b in range(_NBUF):
                do_chunk(t * _NBUF + b, b)
            return carry

        lax.fori_loop(0, chunks_per_worker // _NBUF, loop_body, 0)

        for b in range(_NBUF):
            last = row0 + (chunks_per_worker - _NBUF + b) * _CHUNK
            pltpu.make_async_copy(
                rows_v.at[b], out_hbm.at[pl.ds(last, _CHUNK)], osems[b]
            ).wait()

    return gather_kernel


def kernel(text, table):
    batch, seq = text.shape
    vocab, dim = table.shape
    n_rows = batch * seq
    idx2d = text.reshape(n_rows // _LANES, _LANES).astype(jnp.int32)
    out = _build(n_rows, vocab, dim)(idx2d, table)
    return out.reshape(batch, seq, dim)
